# baseline (speedup 1.0000x reference)
import functools

import jax
import jax.numpy as jnp
from jax import lax
from jax.experimental import pallas as pl
from jax.experimental.pallas import tpu as pltpu
from jax.experimental.pallas import tpu_sc as plsc

N = 10000
E = 320000
D = 128
C = 4
BN_EPS = 1e-5

NC = 2
NS = 16
NW = NC * NS
EPT = E // NW
K = 80
NCHUNK = EPT // K
RPT = 624
REM = N - NS * RPT
LANES = 16
G = D // LANES
G2 = D // 32
DH = D // 2


def _sc_edge_body(x_hbm, src_hbm, dst_hbm, ea_hbm, out_hbm, aggr_sh,
                  xb0, xb1, xb2, eb0, eb1, eb2, sv0, sv1, sv2,
                  dv0, dv1, dv2, dv3, dv4, dv5,
                  gs0, gs1, gs2, es0, es1, es2, ss0, ss1, ss2,
                  is0, is1, is2, ds0, ds1, ds2, ds3, ds4, ds5):
    xb = (xb0, xb1, xb2)
    eb = (eb0, eb1, eb2)
    sv = (sv0, sv1, sv2)
    dv = (dv0, dv1, dv2, dv3, dv4, dv5)
    gs = (gs0, gs1, gs2)
    es = (es0, es1, es2)
    ss = (ss0, ss1, ss2)
    isem = (is0, is1, is2)
    dsem = (ds0, ds1, ds2, ds3, ds4, ds5)
    c = lax.axis_index("c")
    s = lax.axis_index("s")
    wid = c * NS + s
    tile_base = wid * EPT

    def zrow(r, carry):
        for g in range(G):
            xb0[r, pl.ds(LANES * g, LANES)] = jnp.zeros((LANES,), jnp.float32)
        return carry
    lax.fori_loop(0, K, zrow, 0)
    for j in range(RPT // K):
        pltpu.sync_copy(xb0, aggr_sh.at[pl.ds(s * RPT + j * K, K)])
    ZREM = RPT - (RPT // K) * K
    pltpu.sync_copy(xb0.at[pl.ds(0, ZREM)],
                    aggr_sh.at[pl.ds(s * RPT + (RPT // K) * K, ZREM)])

    @pl.when(s == NS - 1)
    def _zero_rem():
        pltpu.sync_copy(xb0.at[pl.ds(0, REM)], aggr_sh.at[pl.ds(NS * RPT, REM)])
    plsc.subcore_barrier()

    def issue_idx(i, j3, j6):
        @pl.when(i < NCHUNK)
        def _():
            base = tile_base + i * K
            pltpu.async_copy(src_hbm.at[pl.ds(base, K)], sv[j3], isem[j3])
            pltpu.async_copy(dst_hbm.at[pl.ds(base, K)], dv[j6], dsem[j6])

    def issue_data(i, j3, guard):
        @pl.when(i < NCHUNK)
        def _():
            pltpu.make_async_copy(src_hbm.at[pl.ds(0, K)], sv[j3],
                                  isem[j3]).wait()
            if guard:
                pltpu.make_async_copy(xb[j3], aggr_sh.at[dv[0]],
                                      ss[j3]).wait()
            pltpu.async_copy(x_hbm.at[sv[j3]], xb[j3], gs[j3])
            base2 = wid * (EPT // 2) + i * (K // 2)
            pltpu.async_copy(ea_hbm.at[pl.ds(base2, K // 2)], eb[j3], es[j3])

    def consume(i, j3, j6):
        pltpu.make_async_copy(x_hbm.at[sv[j3]], xb[j3], gs[j3]).wait()
        pltpu.make_async_copy(ea_hbm.at[pl.ds(0, K // 2)], eb[j3],
                              es[j3]).wait()

        def rowpair(rr, rcarry):
            for p in range(2):
                r = 2 * rr + p
                for g in range(G2):
                    ew = eb[j3][rr, pl.ds(64 * p + LANES * g, LANES)]
                    elo = lax.bitcast_convert_type(
                        jnp.left_shift(ew, 16), jnp.float32)
                    ehi = lax.bitcast_convert_type(
                        jnp.bitwise_and(ew, jnp.int32(-65536)), jnp.float32)
                    a = pl.ds(32 * g, LANES)
                    b = pl.ds(32 * g + LANES, LANES)
                    xb[j3][r, a] = jnp.maximum(xb[j3][r, a] + elo, 0.0)
                    xb[j3][r, b] = jnp.maximum(xb[j3][r, b] + ehi, 0.0)
            return rcarry
        lax.fori_loop(0, K // 2, rowpair, 0)
        pltpu.make_async_copy(dst_hbm.at[pl.ds(0, K)], dv[j6], dsem[j6]).wait()
        pltpu.async_copy(xb[j3], aggr_sh.at[dv[j6]], ss[j3], add=True)

    def step(i, j3, j6, guard):
        issue_data(i + 1, (j3 + 1) % 3, guard)
        consume(i, j3, j6)
        issue_idx(i + 3, j3, (j6 + 3) % 6)

    issue_idx(0, 0, 0)
    issue_idx(1, 1, 1)
    issue_idx(2, 2, 2)
    issue_data(0, 0, False)

    step(0, 0, 0, False)
    step(1, 1, 1, False)
    step(2, 2, 2, True)
    step(3, 0, 3, True)
    step(4, 1, 4, True)

    def six(t, carry):
        i = 6 * t
        step(i + 5, 2, 5, True)
        step(i + 6, 0, 0, True)
        step(i + 7, 1, 1, True)
        step(i + 8, 2, 2, True)
        step(i + 9, 0, 3, True)
        step(i + 10, 1, 4, True)
        return carry
    lax.fori_loop(0, (NCHUNK - 5) // 6, six, 0)
    for j3 in (0, 1):
        pltpu.make_async_copy(xb[j3], aggr_sh.at[dv[0]], ss[j3]).wait()

    plsc.subcore_barrier()
    pltpu.sync_copy(aggr_sh.at[pl.ds(s * RPT, RPT)],
                    out_hbm.at[c, pl.ds(s * RPT, RPT)])

    @pl.when(s == NS - 1)
    def _copy_rem():
        pltpu.sync_copy(aggr_sh.at[pl.ds(NS * RPT, REM)],
                        out_hbm.at[c, pl.ds(NS * RPT, REM)])


_sc_edge = functools.partial(
    pl.kernel,
    mesh=plsc.VectorSubcoreMesh(core_axis_name="c", subcore_axis_name="s"),
    out_type=jax.ShapeDtypeStruct((NC, N, D), jnp.float32),
    scratch_types=(
        [pltpu.VMEM_SHARED((N, D), jnp.float32)]
        + [pltpu.VMEM((K, D), jnp.float32)] * 3
        + [pltpu.VMEM((K // 2, D), jnp.int32)] * 3
        + [pltpu.VMEM((K,), jnp.int32)] * 3
        + [pltpu.VMEM((K,), jnp.int32)] * 6
        + [pltpu.SemaphoreType.DMA] * 18
    ),
)(_sc_edge_body)


def _tc_body(final, x_ref, aggr_ref, w1_ref, b1_ref, w2_ref, b2_ref,
             mask_ref, gamma_ref, beta_ref, xin_ref, eps_ref, out_ref):
    x = x_ref[...]
    a = aggr_ref[0] + aggr_ref[1]
    h = (1.0 + eps_ref[0, 0]) * x + a
    h = jnp.maximum(jnp.dot(h, w1_ref[...],
                            preferred_element_type=jnp.float32) + b1_ref[...], 0.0)
    h = jnp.dot(h, w2_ref[...], preferred_element_type=jnp.float32) + b2_ref[...]
    y = mask_ref[...] * h + x
    mu = jnp.mean(y, axis=0, keepdims=True)
    var = jnp.mean((y - mu) * (y - mu), axis=0, keepdims=True)
    y = gamma_ref[...] * (y - mu) * lax.rsqrt(var + BN_EPS) + beta_ref[...]
    if final:
        y = xin_ref[...] + jnp.maximum(y, 0.0)
    out_ref[...] = y


def _tc_update(x, aggr2, w1, b1, w2, b2, mask, gamma, beta, x_in, eps_c, final):
    return pl.pallas_call(
        functools.partial(_tc_body, final),
        out_shape=jax.ShapeDtypeStruct((N, D), jnp.float32),
        in_specs=[pl.BlockSpec(memory_space=pltpu.VMEM)] * 10
        + [pl.BlockSpec(memory_space=pltpu.SMEM)],
    )(x, aggr2, w1, b1, w2, b2, mask, gamma, beta, x_in, eps_c)


def _perm16(a):
    m = a.shape[0]
    a16 = a.astype(jnp.bfloat16)
    a16 = a16.reshape(m, G2, 2, LANES).swapaxes(2, 3)
    return jax.lax.bitcast_convert_type(a16, jnp.int32).reshape(m, DH)


def kernel(x, edge_index, edge_attr, masks, complement_masks,
           W1, b1, W2, b2, eps, gamma, beta):
    src = edge_index[0]
    dst = edge_index[1]
    ea16 = _perm16(edge_attr).reshape(E // 2, D)
    x_in = x
    for c in range(C):
        aggr2 = _sc_edge(x, src, dst, ea16)
        x = _tc_update(
            x, aggr2, W1[c], b1[c].reshape(1, D), W2[c], b2[c].reshape(1, D),
            masks[c].reshape(N, 1), gamma[c].reshape(1, D), beta[c].reshape(1, D),
            x_in, eps[c].reshape(1, 1), final=(c == C - 1))
    return x


# speedup vs baseline: 1.0099x; 1.0099x over previous
import functools

import jax
import jax.numpy as jnp
from jax import lax
from jax.experimental import pallas as pl
from jax.experimental.pallas import tpu as pltpu
from jax.experimental.pallas import tpu_sc as plsc

N = 10000
E = 320000
D = 128
C = 4
BN_EPS = 1e-5

NC = 2
NS = 16
NW = NC * NS
EPT = E // NW
K = 80
NCHUNK = EPT // K
RPT = 624
REM = N - NS * RPT
LANES = 16
G = D // LANES
G2 = D // 32
DH = D // 2


def _sc_edge_body(x_hbm, src_hbm, dst_hbm, ea_hbm, out_hbm, aggr_sh,
                  xb0, xb1, xb2, eb0, eb1, eb2, sv0, sv1, sv2,
                  dv0, dv1, dv2,
                  gs0, gs1, gs2, es0, es1, es2, ss0, ss1, ss2,
                  is0, is1, is2, ds0, ds1, ds2):
    xb = (xb0, xb1, xb2)
    eb = (eb0, eb1, eb2)
    sv = (sv0, sv1, sv2)
    dv = (dv0, dv1, dv2)
    gs = (gs0, gs1, gs2)
    es = (es0, es1, es2)
    ss = (ss0, ss1, ss2)
    isem = (is0, is1, is2)
    dsem = (ds0, ds1, ds2)
    c = lax.axis_index("c")
    s = lax.axis_index("s")
    wid = c * NS + s
    tile_base = wid * EPT

    def zrow(r, carry):
        for g in range(G):
            xb0[r, pl.ds(LANES * g, LANES)] = jnp.zeros((LANES,), jnp.float32)
        return carry
    lax.fori_loop(0, K, zrow, 0)
    for j in range(RPT // K):
        pltpu.sync_copy(xb0, aggr_sh.at[pl.ds(s * RPT + j * K, K)])
    ZREM = RPT - (RPT // K) * K
    pltpu.sync_copy(xb0.at[pl.ds(0, ZREM)],
                    aggr_sh.at[pl.ds(s * RPT + (RPT // K) * K, ZREM)])

    @pl.when(s == NS - 1)
    def _zero_rem():
        pltpu.sync_copy(xb0.at[pl.ds(0, REM)], aggr_sh.at[pl.ds(NS * RPT, REM)])
    plsc.subcore_barrier()

    def issue_idx(i, j3):
        @pl.when(i < NCHUNK)
        def _():
            base = tile_base + i * K
            pltpu.async_copy(src_hbm.at[pl.ds(base, K)], sv[j3], isem[j3])
            pltpu.async_copy(dst_hbm.at[pl.ds(base, K)], dv[j3], dsem[j3])

    def issue_data(i, j3, guard):
        @pl.when(i < NCHUNK)
        def _():
            pltpu.make_async_copy(src_hbm.at[pl.ds(0, K)], sv[j3],
                                  isem[j3]).wait()
            if guard:
                pltpu.make_async_copy(xb[j3], aggr_sh.at[dv[0]],
                                      ss[j3]).wait()
            pltpu.async_copy(x_hbm.at[sv[j3]], xb[j3], gs[j3])
            base2 = wid * (EPT // 2) + i * (K // 2)
            pltpu.async_copy(ea_hbm.at[pl.ds(base2, K // 2)], eb[j3], es[j3])

    def consume(i, j3):
        pltpu.make_async_copy(x_hbm.at[sv[j3]], xb[j3], gs[j3]).wait()
        pltpu.make_async_copy(ea_hbm.at[pl.ds(0, K // 2)], eb[j3],
                              es[j3]).wait()

        def row(r, rcarry):
            rr = r // 2
            wb = 64 * (r % 2)
            for g in range(G2):
                ew = eb[j3][rr, pl.ds(wb + LANES * g, LANES)]
                elo = lax.bitcast_convert_type(
                    jnp.left_shift(ew, 16), jnp.float32)
                ehi = lax.bitcast_convert_type(
                    jnp.bitwise_and(ew, jnp.int32(-65536)), jnp.float32)
                a = pl.ds(32 * g, LANES)
                b = pl.ds(32 * g + LANES, LANES)
                xb[j3][r, a] = jnp.maximum(xb[j3][r, a] + elo, 0.0)
                xb[j3][r, b] = jnp.maximum(xb[j3][r, b] + ehi, 0.0)
            return rcarry
        lax.fori_loop(0, K, row, 0)
        pltpu.make_async_copy(dst_hbm.at[pl.ds(0, K)], dv[j3], dsem[j3]).wait()
        pltpu.async_copy(xb[j3], aggr_sh.at[dv[j3]], ss[j3], add=True)

    def step(i, j3, guard):
        issue_data(i + 1, (j3 + 1) % 3, guard)
        consume(i, j3)
        issue_idx(i + 3, j3)

    issue_idx(0, 0)
    issue_idx(1, 1)
    issue_idx(2, 2)
    issue_data(0, 0, False)

    step(0, 0, False)
    step(1, 1, False)

    def three(t, carry):
        i = 3 * t
        step(i + 2, 2, True)
        step(i + 3, 0, True)
        step(i + 4, 1, True)
        return carry
    lax.fori_loop(0, (NCHUNK - 2) // 3, three, 0)
    for j3 in (0, 1):
        pltpu.make_async_copy(xb[j3], aggr_sh.at[dv[0]], ss[j3]).wait()

    plsc.subcore_barrier()
    pltpu.sync_copy(aggr_sh.at[pl.ds(s * RPT, RPT)],
                    out_hbm.at[c, pl.ds(s * RPT, RPT)])

    @pl.when(s == NS - 1)
    def _copy_rem():
        pltpu.sync_copy(aggr_sh.at[pl.ds(NS * RPT, REM)],
                        out_hbm.at[c, pl.ds(NS * RPT, REM)])


_sc_edge = functools.partial(
    pl.kernel,
    mesh=plsc.VectorSubcoreMesh(core_axis_name="c", subcore_axis_name="s"),
    out_type=jax.ShapeDtypeStruct((NC, N, D), jnp.float32),
    scratch_types=(
        [pltpu.VMEM_SHARED((N, D), jnp.float32)]
        + [pltpu.VMEM((K, D), jnp.float32)] * 3
        + [pltpu.VMEM((K // 2, D), jnp.int32)] * 3
        + [pltpu.VMEM((K,), jnp.int32)] * 3
        + [pltpu.VMEM((K,), jnp.int32)] * 3
        + [pltpu.SemaphoreType.DMA] * 15
    ),
)(_sc_edge_body)


def _tc_body(final, x_ref, aggr_ref, w1_ref, b1_ref, w2_ref, b2_ref,
             mask_ref, gamma_ref, beta_ref, xin_ref, eps_ref, out_ref):
    x = x_ref[...]
    a = aggr_ref[0] + aggr_ref[1]
    h = (1.0 + eps_ref[0, 0]) * x + a
    h = jnp.maximum(jnp.dot(h, w1_ref[...],
                            preferred_element_type=jnp.float32) + b1_ref[...], 0.0)
    h = jnp.dot(h, w2_ref[...], preferred_element_type=jnp.float32) + b2_ref[...]
    y = mask_ref[...] * h + x
    mu = jnp.mean(y, axis=0, keepdims=True)
    var = jnp.mean((y - mu) * (y - mu), axis=0, keepdims=True)
    y = gamma_ref[...] * (y - mu) * lax.rsqrt(var + BN_EPS) + beta_ref[...]
    if final:
        y = xin_ref[...] + jnp.maximum(y, 0.0)
    out_ref[...] = y


def _tc_update(x, aggr2, w1, b1, w2, b2, mask, gamma, beta, x_in, eps_c, final):
    return pl.pallas_call(
        functools.partial(_tc_body, final),
        out_shape=jax.ShapeDtypeStruct((N, D), jnp.float32),
        in_specs=[pl.BlockSpec(memory_space=pltpu.VMEM)] * 10
        + [pl.BlockSpec(memory_space=pltpu.SMEM)],
    )(x, aggr2, w1, b1, w2, b2, mask, gamma, beta, x_in, eps_c)


def _perm16(a):
    m = a.shape[0]
    a16 = a.astype(jnp.bfloat16)
    a16 = a16.reshape(m, G2, 2, LANES).swapaxes(2, 3)
    return jax.lax.bitcast_convert_type(a16, jnp.int32).reshape(m, DH)


def kernel(x, edge_index, edge_attr, masks, complement_masks,
           W1, b1, W2, b2, eps, gamma, beta):
    src = edge_index[0]
    dst = edge_index[1]
    ea16 = _perm16(edge_attr).reshape(E // 2, D)
    x_in = x
    for c in range(C):
        aggr2 = _sc_edge(x, src, dst, ea16)
        x = _tc_update(
            x, aggr2, W1[c], b1[c].reshape(1, D), W2[c], b2[c].reshape(1, D),
            masks[c].reshape(N, 1), gamma[c].reshape(1, D), beta[c].reshape(1, D),
            x_in, eps[c].reshape(1, 1), final=(c == C - 1))
    return x


# speedup vs baseline: 2.6453x; 2.6193x over previous
import functools

import jax
import jax.numpy as jnp
from jax import lax
from jax.experimental import pallas as pl
from jax.experimental.pallas import tpu as pltpu
from jax.experimental.pallas import tpu_sc as plsc

N = 10000
E = 320000
D = 128
C = 4
BN_EPS = 1e-5

NC = 2
NS = 16
NW = NC * NS
EPT = E // NW
K = 40
NCHUNK = EPT // K
NB = 3
RPT = 624
REM = N - NS * RPT
LANES = 16
G = D // LANES


def _sc_edge_body(x_hbm, src_hbm, dst_hbm, ea_hbm, out_hbm, aggr_sh, src_t,
                  xb0, xb1, xb2, eb0, eb1, eb2, dv0, dv1, dv2,
                  gs0, gs1, gs2, es0, es1, es2, is0, is1, is2,
                  ss0, ss1, ss2):
    xb = (xb0, xb1, xb2)
    eb = (eb0, eb1, eb2)
    dv = (dv0, dv1, dv2)
    gs = (gs0, gs1, gs2)
    es = (es0, es1, es2)
    isem = (is0, is1, is2)
    ss = (ss0, ss1, ss2)
    c = lax.axis_index("c")
    s = lax.axis_index("s")
    wid = c * NS + s
    tile_base = wid * EPT

    def zrow(r, carry):
        for g in range(G):
            eb0[r, pl.ds(LANES * g, LANES)] = jnp.zeros((LANES,), jnp.float32)
        return carry
    lax.fori_loop(0, K, zrow, 0)
    for j in range(RPT // K):
        pltpu.sync_copy(eb0, aggr_sh.at[pl.ds(s * RPT + j * K, K)])
    ZREM = RPT - (RPT // K) * K
    pltpu.sync_copy(eb0.at[pl.ds(0, ZREM)],
                    aggr_sh.at[pl.ds(s * RPT + (RPT // K) * K, ZREM)])

    @pl.when(s == NS - 1)
    def _zero_rem():
        pltpu.sync_copy(eb0.at[pl.ds(0, REM)], aggr_sh.at[pl.ds(NS * RPT, REM)])

    pltpu.sync_copy(src_hbm.at[pl.ds(tile_base, EPT)], src_t)
    plsc.subcore_barrier()

    def issue(i, j, guard):
        @pl.when(i < NCHUNK)
        def _():
            if guard:
                pltpu.make_async_copy(eb[j], aggr_sh.at[dv[j]], ss[j]).wait()
            base = tile_base + i * K
            pltpu.async_copy(dst_hbm.at[pl.ds(base, K)], dv[j], isem[j])
            pltpu.async_copy(ea_hbm.at[pl.ds(base, K)], eb[j], es[j])
            pltpu.async_copy(x_hbm.at[src_t.at[pl.ds(i * K, K)]], xb[j], gs[j])

    def consume(i, j):
        base = tile_base + i * K
        pltpu.make_async_copy(x_hbm.at[src_t.at[pl.ds(i * K, K)]],
                              xb[j], gs[j]).wait()
        pltpu.make_async_copy(ea_hbm.at[pl.ds(base, K)], eb[j], es[j]).wait()
        pltpu.make_async_copy(dst_hbm.at[pl.ds(base, K)], dv[j], isem[j]).wait()

        def row(r, rcarry):
            for g in range(G):
                sl = pl.ds(LANES * g, LANES)
                eb[j][r, sl] = jnp.maximum(xb[j][r, sl] + eb[j][r, sl], 0.0)
            return rcarry
        lax.fori_loop(0, K, row, 0)
        pltpu.async_copy(eb[j], aggr_sh.at[dv[j]], ss[j], add=True)

    issue(0, 0, False)
    issue(1, 1, False)
    issue(2, 2, False)
    consume(0, 0)
    issue(3, 0, True)
    consume(1, 1)
    issue(4, 1, True)
    consume(2, 2)
    issue(5, 2, True)

    def block(t, carry):
        i = 3 * t
        for k in range(3):
            consume(i + k, k)
            issue(i + k + 3, k, True)
        return carry
    lax.fori_loop(1, NCHUNK // 3, block, 0)
    consume(NCHUNK - 1, 0)
    for j in range(NB):
        pltpu.make_async_copy(eb[j], aggr_sh.at[dv[j]], ss[j]).wait()

    plsc.subcore_barrier()
    pltpu.sync_copy(aggr_sh.at[pl.ds(s * RPT, RPT)],
                    out_hbm.at[c, pl.ds(s * RPT, RPT)])

    @pl.when(s == NS - 1)
    def _copy_rem():
        pltpu.sync_copy(aggr_sh.at[pl.ds(NS * RPT, REM)],
                        out_hbm.at[c, pl.ds(NS * RPT, REM)])


_sc_edge = functools.partial(
    pl.kernel,
    mesh=plsc.VectorSubcoreMesh(core_axis_name="c", subcore_axis_name="s"),
    out_type=jax.ShapeDtypeStruct((NC, N, D), jnp.float32),
    scratch_types=(
        [pltpu.VMEM_SHARED((N, D), jnp.float32)]
        + [pltpu.VMEM((EPT,), jnp.int32)]
        + [pltpu.VMEM((K, D), jnp.float32)] * 3
        + [pltpu.VMEM((K, D), jnp.float32)] * 3
        + [pltpu.VMEM((K,), jnp.int32)] * 3
        + [pltpu.SemaphoreType.DMA] * 12
    ),
)(_sc_edge_body)


def _tc_body(final, x_ref, aggr_ref, w1_ref, b1_ref, w2_ref, b2_ref,
             mask_ref, gamma_ref, beta_ref, xin_ref, eps_ref, out_ref):
    x = x_ref[...]
    a = aggr_ref[0] + aggr_ref[1]
    h = (1.0 + eps_ref[0, 0]) * x + a
    h = jnp.maximum(jnp.dot(h, w1_ref[...],
                            preferred_element_type=jnp.float32) + b1_ref[...], 0.0)
    h = jnp.dot(h, w2_ref[...], preferred_element_type=jnp.float32) + b2_ref[...]
    y = mask_ref[...] * h + x
    mu = jnp.mean(y, axis=0, keepdims=True)
    var = jnp.mean((y - mu) * (y - mu), axis=0, keepdims=True)
    y = gamma_ref[...] * (y - mu) * lax.rsqrt(var + BN_EPS) + beta_ref[...]
    if final:
        y = xin_ref[...] + jnp.maximum(y, 0.0)
    out_ref[...] = y


def _tc_update(x, aggr2, w1, b1, w2, b2, mask, gamma, beta, x_in, eps_c, final):
    return pl.pallas_call(
        functools.partial(_tc_body, final),
        out_shape=jax.ShapeDtypeStruct((N, D), jnp.float32),
        in_specs=[pl.BlockSpec(memory_space=pltpu.VMEM)] * 10
        + [pl.BlockSpec(memory_space=pltpu.SMEM)],
    )(x, aggr2, w1, b1, w2, b2, mask, gamma, beta, x_in, eps_c)


def kernel(x, edge_index, edge_attr, masks, complement_masks,
           W1, b1, W2, b2, eps, gamma, beta):
    src = edge_index[0]
    dst = edge_index[1]
    x_in = x
    for c in range(C):
        aggr2 = _sc_edge(x, src, dst, edge_attr)
        x = _tc_update(
            x, aggr2, W1[c], b1[c].reshape(1, D), W2[c], b2[c].reshape(1, D),
            masks[c].reshape(N, 1), gamma[c].reshape(1, D), beta[c].reshape(1, D),
            x_in, eps[c].reshape(1, 1), final=(c == C - 1))
    return x
